# Initial kernel scaffold; baseline (speedup 1.0000x reference)
import jax
import jax.numpy as jnp
from jax.experimental import pallas as pl


def kernel(x, edge_index, node_type, params):
    raise NotImplementedError("write your pallas kernel here")


# speedup vs baseline: 1.0549x; 1.0549x over previous
import functools

import jax
import jax.numpy as jnp
from jax import lax
from jax.experimental import pallas as pl
from jax.experimental.pallas import tpu as pltpu
from jax.experimental.pallas import tpu_sc as plsc

H = 64
HALF = 5000
N = 10000
NC = 2
NS = 16
NW = NC * NS
K = 128
GROUPS = K // 16
ROWS_PER_TILE = N // NS


def _lnk(x, g, b):
    m = jnp.mean(x, axis=-1, keepdims=True)
    v = jnp.mean((x - m) ** 2, axis=-1, keepdims=True)
    return (x - m) / jnp.sqrt(v + 1e-5) * g + b


def _dotT(x, w):
    return lax.dot_general(x, w, (((1,), (1,)), ((), ())),
                           preferred_element_type=jnp.float32)


def _gru_k(agg, h, wi, bi, wh, bh):
    gi = _dotT(agg, wi) + bi
    gh = _dotT(h, wh) + bh
    r = jax.nn.sigmoid(gi[:, :H] + gh[:, :H])
    z = jax.nn.sigmoid(gi[:, H:2 * H] + gh[:, H:2 * H])
    n = jnp.tanh(gi[:, 2 * H:] + r * gh[:, 2 * H:])
    return (1.0 - z) * n + z * h


def _mlp2_k(x, w1, b1, g1, be1, w2, b2, g2, be2):
    h1 = jax.nn.relu(_lnk(_dotT(x, w1) + b1, g1, be1))
    return _lnk(_dotT(h1, w2) + b2, g2, be2)


def _embed_body(x_r, wv_r, eb_r, eg_r, ebe_r,
                w1_r, b1_r, g1_r, be1_r, w2_r, b2_r, g2_r, be2_r,
                w1a_r, w1b_r, ab1_r,
                h_r, a_r, mb_r):
    pre = x_r[...] * wv_r[...] + eb_r[...]
    h0 = jax.nn.relu(_lnk(pre, eg_r[...], ebe_r[...]))
    m = _mlp2_k(h0, w1_r[0], b1_r[0], g1_r[0], be1_r[0],
                w2_r[0], b2_r[0], g2_r[0], be2_r[0])
    h_r[...] = h0
    a = _dotT(h0, w1a_r[...])
    a_r[:, :H] = a
    a_r[:, H:] = a
    mb_r[:, :H] = m
    mb_r[:, H:] = _dotT(m, w1b_r[...]) + ab1_r[...]


def _update_body(agg2_r, h_r,
                 wi_r, bi_r, wh_r, bh_r, lg_r, lb_r,
                 w1_r, b1_r, g1_r, be1_r, w2_r, b2_r, g2_r, be2_r,
                 w1a_r, w1b_r, ab1_r,
                 hn_r, a_r, mb_r):
    agg = agg2_r[0] + agg2_r[1]
    h = h_r[...]
    hn = _lnk(_gru_k(agg, h, wi_r[0], bi_r[0], wh_r[0], bh_r[0]) + h,
              lg_r[0], lb_r[0])
    m = _mlp2_k(hn, w1_r[0], b1_r[0], g1_r[0], be1_r[0],
                w2_r[0], b2_r[0], g2_r[0], be2_r[0])
    hn_r[...] = hn
    a = _dotT(hn, w1a_r[...])
    a_r[:, :H] = a
    a_r[:, H:] = a
    mb_r[:, :H] = m
    mb_r[:, H:] = _dotT(m, w1b_r[...]) + ab1_r[...]


def _final_body(agg2_r, h_r,
                wi_r, bi_r, wh_r, bh_r, lg_r, lb_r,
                gw1_r, gb1_r, gw2_r, gb2_r,
                aw1_r, ab1_r, ag_r, abe_r, aw2_r, ab2_r,
                vw1_r, vb1_r, vg_r, vbe_r, vw2_r, vb2_r,
                logits_r, value_r):
    agg = agg2_r[0] + agg2_r[1]
    h = h_r[...]
    vf = _lnk(_gru_k(agg, h, wi_r[...], bi_r[...], wh_r[...], bh_r[...]) + h,
              lg_r[...], lb_r[...])
    gw = jax.nn.sigmoid(
        jnp.sum(jax.nn.relu(_dotT(vf, gw1_r[...]) + gb1_r[...]) * gw2_r[...],
                axis=-1, keepdims=True) + gb2_r[...])
    glob = jnp.sum(vf * gw, axis=0, keepdims=True)
    ha = jax.nn.relu(_lnk(_dotT(vf, aw1_r[...]) + ab1_r[...],
                          ag_r[...], abe_r[...]))
    logits_r[...] = _dotT(ha, aw2_r[...]) + ab2_r[...]
    hv = jax.nn.relu(_lnk(_dotT(glob, vw1_r[...]) + vb1_r[...],
                          vg_r[...], vbe_r[...]))
    value_r[...] = (jnp.sum(hv * vw2_r[...], axis=-1, keepdims=True)
                    + vb2_r[...])


def _half_spec(shape):
    return pl.BlockSpec((1,) + shape[1:], lambda i: (i,) + (0,) * (len(shape) - 1))


def _full_spec(shape):
    return pl.BlockSpec(shape, lambda i: (0,) * len(shape))


def _row_spec(shape):
    return pl.BlockSpec((HALF,) + shape[1:], lambda i: (i,) + (0,) * (len(shape) - 1))


def _tc_embed(x2, emb, mlp_s, att, interpret=False):
    in_specs = ([_row_spec((N, 1))] + [_full_spec((1, H))] * 4
                + [_half_spec(s.shape) for s in mlp_s]
                + [_full_spec((H, H)), _full_spec((H, H)), _full_spec((1, H))])
    out_specs = [_row_spec((N, H)), _row_spec((N, 2 * H)), _row_spec((N, 2 * H))]
    out_shape = [jax.ShapeDtypeStruct((N, H), jnp.float32),
                 jax.ShapeDtypeStruct((N, 2 * H), jnp.float32),
                 jax.ShapeDtypeStruct((N, 2 * H), jnp.float32)]
    return pl.pallas_call(
        _embed_body, grid=(2,), in_specs=in_specs, out_specs=out_specs,
        out_shape=out_shape, interpret=interpret,
    )(x2, *emb, *mlp_s, *att)


def _tc_update(agg2, h, gru_s, mlp_s, att, interpret=False):
    in_specs = ([pl.BlockSpec((2, HALF, H), lambda i: (0, i, 0)),
                 _row_spec((N, H))]
                + [_half_spec(s.shape) for s in gru_s]
                + [_half_spec(s.shape) for s in mlp_s]
                + [_full_spec((H, H)), _full_spec((H, H)), _full_spec((1, H))])
    out_specs = [_row_spec((N, H)), _row_spec((N, 2 * H)), _row_spec((N, 2 * H))]
    out_shape = [jax.ShapeDtypeStruct((N, H), jnp.float32),
                 jax.ShapeDtypeStruct((N, 2 * H), jnp.float32),
                 jax.ShapeDtypeStruct((N, 2 * H), jnp.float32)]
    return pl.pallas_call(
        _update_body, grid=(2,), in_specs=in_specs, out_specs=out_specs,
        out_shape=out_shape, interpret=interpret,
    )(agg2, h, *gru_s, *mlp_s, *att)


def _tc_final(agg2v, hv, gru_v, ga, act, val, interpret=False):
    out_shape = [jax.ShapeDtypeStruct((HALF, 4), jnp.float32),
                 jax.ShapeDtypeStruct((1, 1), jnp.float32)]
    return pl.pallas_call(
        _final_body, out_shape=out_shape, interpret=interpret,
    )(agg2v, hv, *gru_v, *ga, *act, *val)


def _edge_body(src_hbm, dst_hbm, a_hbm, mb_hbm, w2_hbm, b2_hbm, out_hbm,
               sidx, didx, mb_v, a_v, con_v, w2_v,
               b2_v, zb_v, acc, sem1, sem2):
    cid = lax.axis_index("c")
    tid = lax.axis_index("s")
    wid = tid * NC + cid
    ept = src_hbm.shape[2]
    nblocks = ept // K

    zrow = jnp.zeros((16,), jnp.float32)
    zrows = zb_v.shape[0]

    def zb_body(i, c):
        for j in range(H // 16):
            zb_v[i, pl.ds(j * 16, 16)] = zrow
        return c
    lax.fori_loop(0, zrows, zb_body, 0)
    base = tid * ROWS_PER_TILE
    for j in range(ROWS_PER_TILE // zrows):
        pltpu.sync_copy(zb_v, acc.at[pl.ds(base + j * zrows, zrows)])
    plsc.subcore_barrier()

    pltpu.sync_copy(w2_hbm, w2_v)
    pltpu.sync_copy(b2_hbm, b2_v)

    w2r = [w2_v[pl.ds(j * 16, 16)] for j in range(H // 16)]
    b2r = b2_v[...]

    def block_body(b, c):
        eb = b * K
        pltpu.sync_copy(src_hbm.at[wid, 0, pl.ds(eb, K)], sidx)
        pltpu.sync_copy(dst_hbm.at[wid, 0, pl.ds(eb, K)], didx)

        for g in range(GROUPS):
            gb = g * 16
            s16 = sidx[pl.ds(gb, 16)]
            d16 = didx[pl.ds(gb, 16)]
            validv = jnp.where((s16 >= HALF) != (d16 >= HALF), 1.0, 0.0)
            for j in range(16):
                e = gb + j
                s = (jnp.maximum(a_v[e, pl.ds(0, 16)]
                                 + mb_v[e, pl.ds(H, 16)], 0.0) * w2r[0])
                for q in range(1, H // 16):
                    s = s + (jnp.maximum(a_v[e, pl.ds(q * 16, 16)]
                                         + mb_v[e, pl.ds(H + q * 16, 16)], 0.0)
                             * w2r[q])
                logit = jnp.full((16,), lax.reduce_sum(s, (0,)), jnp.float32)
                att = validv[j] / (1.0 + jnp.exp(-(logit + b2r)))
                for q in range(H // 16):
                    con_v[e, pl.ds(q * 16, 16)] = (
                        mb_v[e, pl.ds(q * 16, 16)] * att)
        pltpu.sync_copy(con_v, acc.at[didx], add=True)
        return c
    lax.fori_loop(0, nblocks, block_body, 0)

    plsc.subcore_barrier()
    pltpu.sync_copy(acc.at[pl.ds(base, ROWS_PER_TILE)], out_hbm.at[cid, tid])


def _sc_edge(src_p, dst_p, a_t, mb_t, w2, b2bc):
    src = src_p.reshape(-1)
    dst = dst_p.reshape(-1)
    m = mb_t[:, :H]
    b = mb_t[:, H:]
    pre = a_t[:, :H][dst] + b[src]
    logit = jax.nn.relu(pre) @ w2 + b2bc[0]
    att = jax.nn.sigmoid(logit)
    valid = ((src >= HALF) != (dst >= HALF)).astype(jnp.float32)
    contrib = m[src] * (att * valid)[:, None]
    agg = jnp.zeros((N, H), jnp.float32).at[dst].add(contrib)
    return jnp.stack([agg, jnp.zeros_like(agg)], axis=0)


def _sc_edge_unused(src_p, dst_p, a_t, mb_t, w2, b2bc):
    ept = src_p.shape[2]
    mesh = plsc.VectorSubcoreMesh(core_axis_name="c", subcore_axis_name="s",
                                  num_cores=NC, num_subcores=NS)
    f = functools.partial(
        pl.kernel, _edge_body, mesh=mesh,
        compiler_params=pltpu.CompilerParams(needs_layout_passes=False),
        out_type=jax.ShapeDtypeStruct((2, NS, ROWS_PER_TILE, H), jnp.float32),
        scratch_types=[
            pltpu.VMEM((K,), jnp.int32),
            pltpu.VMEM((K,), jnp.int32),
            pltpu.VMEM((K, 2 * H), jnp.float32),
            pltpu.VMEM((K, 2 * H), jnp.float32),
            pltpu.VMEM((K, H), jnp.float32),
            pltpu.VMEM((H,), jnp.float32),
            pltpu.VMEM((16,), jnp.float32),
            pltpu.VMEM((25, H), jnp.float32),
            pltpu.VMEM_SHARED((N, H), jnp.float32),
            pltpu.SemaphoreType.DMA,
            pltpu.SemaphoreType.DMA,
        ],
    )()
    return f(src_p, dst_p, a_t, mb_t, w2, b2bc).reshape(2, N, H)


def _stack(layers, *path):
    def get(lp):
        v = lp
        for p in path:
            v = v[p]
        return v
    return jnp.stack([get(lp) for lp in layers], axis=0)


def _row2(v):
    return v.reshape(1, -1)


def kernel(x, edge_index, node_type, params):
    del node_type
    e = edge_index.shape[1]
    ept = -(-e // (NW * K)) * K
    pad = NW * ept - e
    src_p = jnp.pad(edge_index[0], (0, pad)).reshape(NW, 1, ept)
    dst_p = jnp.pad(edge_index[1], (0, pad)).reshape(NW, 1, ept)

    lps = params['layers']
    emb = params['embed']
    emb_in = (_row2(emb['W'][:, 0]), _row2(emb['b']), _row2(emb['g']),
              _row2(emb['be']))

    def mlp_stack(l):
        pair = (lps[l]['c2v'], lps[l]['v2c'])
        out = []
        for k in ('W1', 'b1', 'g1', 'be1', 'W2', 'b2', 'g2', 'be2'):
            v = jnp.stack([p[k] for p in pair], axis=0)
            if v.ndim == 2:
                v = v[:, None, :]
            out.append(v)
        return out

    def att_in(l):
        a = lps[l]['att']
        w1 = a['W1']
        return (w1[:, :H], w1[:, H:], _row2(a['b1']))

    def gru_stack(l):
        pair = ((lps[l]['gru_check'], lps[l]['ln_check']),
                (lps[l]['gru_var'], lps[l]['ln_var']))
        out = []
        for k in ('Wi', 'bi', 'Wh', 'bh'):
            v = jnp.stack([p[0][k] for p in pair], axis=0)
            out.append(v[:, None, :] if v.ndim == 2 else v)
        out.append(jnp.stack([p[1]['g'] for p in pair], axis=0)[:, None, :])
        out.append(jnp.stack([p[1]['b'] for p in pair], axis=0)[:, None, :])
        return out

    h, a_t, mb_t = _tc_embed(x.reshape(N, 1), emb_in, mlp_stack(0), att_in(0))

    for l in range(len(lps)):
        att = lps[l]['att']
        w2 = att['W2'][0]
        b2bc = jnp.broadcast_to(att['b2'], (16,))
        agg2 = _sc_edge(src_p, dst_p, a_t, mb_t, w2, b2bc)
        if l + 1 < len(lps):
            h, a_t, mb_t = _tc_update(agg2, h, gru_stack(l),
                                      mlp_stack(l + 1), att_in(l + 1))
        else:
            gv = lps[l]['gru_var']
            lv = lps[l]['ln_var']
            gru_v = (gv['Wi'], _row2(gv['bi']), gv['Wh'], _row2(gv['bh']),
                     _row2(lv['g']), _row2(lv['b']))
            ga = params['glob_att']
            ga_in = (ga['W1'], _row2(ga['b1']), ga['W2'], ga['b2'].reshape(1, 1))
            ap = params['action']
            act_in = (ap['W1'], _row2(ap['b1']), _row2(ap['g']), _row2(ap['be']),
                      ap['W2'], _row2(ap['b2']))
            vp = params['value']
            val_in = (vp['W1'], _row2(vp['b1']), _row2(vp['g']), _row2(vp['be']),
                      vp['W2'], vp['b2'].reshape(1, 1))
            logits, value = _tc_final(agg2[:, HALF:], h[HALF:], gru_v,
                                      ga_in, act_in, val_in)
    return (logits, value)


# speedup vs baseline: 2.6106x; 2.4747x over previous
import functools

import jax
import jax.numpy as jnp
from jax import lax
from jax.experimental import pallas as pl
from jax.experimental.pallas import tpu as pltpu
from jax.experimental.pallas import tpu_sc as plsc

H = 64
HALF = 5000
N = 10000
NC = 2
NS = 16
NW = NC * NS
K = 128
GROUPS = K // 16
ROWS_PER_TILE = N // NS


def _lnk(x, g, b):
    m = jnp.mean(x, axis=-1, keepdims=True)
    v = jnp.mean((x - m) ** 2, axis=-1, keepdims=True)
    return (x - m) / jnp.sqrt(v + 1e-5) * g + b


def _dotT(x, w):
    return lax.dot_general(x, w, (((1,), (1,)), ((), ())),
                           preferred_element_type=jnp.float32)


def _gru_k(agg, h, wi, bi, wh, bh):
    gi = _dotT(agg, wi) + bi
    gh = _dotT(h, wh) + bh
    r = jax.nn.sigmoid(gi[:, :H] + gh[:, :H])
    z = jax.nn.sigmoid(gi[:, H:2 * H] + gh[:, H:2 * H])
    n = jnp.tanh(gi[:, 2 * H:] + r * gh[:, 2 * H:])
    return (1.0 - z) * n + z * h


def _mlp2_k(x, w1, b1, g1, be1, w2, b2, g2, be2):
    h1 = jax.nn.relu(_lnk(_dotT(x, w1) + b1, g1, be1))
    return _lnk(_dotT(h1, w2) + b2, g2, be2)


def _embed_body(x_r, wv_r, eb_r, eg_r, ebe_r,
                w1_r, b1_r, g1_r, be1_r, w2_r, b2_r, g2_r, be2_r,
                w1a_r, w1b_r, ab1_r,
                h_r, a_r, mb_r):
    pre = x_r[...] * wv_r[...] + eb_r[...]
    h0 = jax.nn.relu(_lnk(pre, eg_r[...], ebe_r[...]))
    m = _mlp2_k(h0, w1_r[0], b1_r[0], g1_r[0], be1_r[0],
                w2_r[0], b2_r[0], g2_r[0], be2_r[0])
    h_r[...] = h0
    a = _dotT(h0, w1a_r[...])
    a_r[:, :H] = a
    a_r[:, H:] = a
    mb_r[:, :H] = m
    mb_r[:, H:] = _dotT(m, w1b_r[...]) + ab1_r[...]


def _update_body(agg2_r, h_r,
                 wi_r, bi_r, wh_r, bh_r, lg_r, lb_r,
                 w1_r, b1_r, g1_r, be1_r, w2_r, b2_r, g2_r, be2_r,
                 w1a_r, w1b_r, ab1_r,
                 hn_r, a_r, mb_r):
    agg = agg2_r[0] + agg2_r[1]
    h = h_r[...]
    hn = _lnk(_gru_k(agg, h, wi_r[0], bi_r[0], wh_r[0], bh_r[0]) + h,
              lg_r[0], lb_r[0])
    m = _mlp2_k(hn, w1_r[0], b1_r[0], g1_r[0], be1_r[0],
                w2_r[0], b2_r[0], g2_r[0], be2_r[0])
    hn_r[...] = hn
    a = _dotT(hn, w1a_r[...])
    a_r[:, :H] = a
    a_r[:, H:] = a
    mb_r[:, :H] = m
    mb_r[:, H:] = _dotT(m, w1b_r[...]) + ab1_r[...]


def _final_body(agg2_r, h_r,
                wi_r, bi_r, wh_r, bh_r, lg_r, lb_r,
                gw1_r, gb1_r, gw2_r, gb2_r,
                aw1_r, ab1_r, ag_r, abe_r, aw2_r, ab2_r,
                vw1_r, vb1_r, vg_r, vbe_r, vw2_r, vb2_r,
                logits_r, value_r):
    agg = agg2_r[0] + agg2_r[1]
    h = h_r[...]
    vf = _lnk(_gru_k(agg, h, wi_r[...], bi_r[...], wh_r[...], bh_r[...]) + h,
              lg_r[...], lb_r[...])
    gw = jax.nn.sigmoid(
        jnp.sum(jax.nn.relu(_dotT(vf, gw1_r[...]) + gb1_r[...]) * gw2_r[...],
                axis=-1, keepdims=True) + gb2_r[...])
    glob = jnp.sum(vf * gw, axis=0, keepdims=True)
    ha = jax.nn.relu(_lnk(_dotT(vf, aw1_r[...]) + ab1_r[...],
                          ag_r[...], abe_r[...]))
    logits_r[...] = _dotT(ha, aw2_r[...]) + ab2_r[...]
    hv = jax.nn.relu(_lnk(_dotT(glob, vw1_r[...]) + vb1_r[...],
                          vg_r[...], vbe_r[...]))
    value_r[...] = (jnp.sum(hv * vw2_r[...], axis=-1, keepdims=True)
                    + vb2_r[...])


def _half_spec(shape):
    return pl.BlockSpec((1,) + shape[1:], lambda i: (i,) + (0,) * (len(shape) - 1))


def _full_spec(shape):
    return pl.BlockSpec(shape, lambda i: (0,) * len(shape))


def _row_spec(shape):
    return pl.BlockSpec((HALF,) + shape[1:], lambda i: (i,) + (0,) * (len(shape) - 1))


def _tc_embed(x2, emb, mlp_s, att, interpret=False):
    in_specs = ([_row_spec((N, 1))] + [_full_spec((1, H))] * 4
                + [_half_spec(s.shape) for s in mlp_s]
                + [_full_spec((H, H)), _full_spec((H, H)), _full_spec((1, H))])
    out_specs = [_row_spec((N, H)), _row_spec((N, 2 * H)), _row_spec((N, 2 * H))]
    out_shape = [jax.ShapeDtypeStruct((N, H), jnp.float32),
                 jax.ShapeDtypeStruct((N, 2 * H), jnp.float32),
                 jax.ShapeDtypeStruct((N, 2 * H), jnp.float32)]
    return pl.pallas_call(
        _embed_body, grid=(2,), in_specs=in_specs, out_specs=out_specs,
        out_shape=out_shape, interpret=interpret,
    )(x2, *emb, *mlp_s, *att)


def _tc_update(agg2, h, gru_s, mlp_s, att, interpret=False):
    in_specs = ([pl.BlockSpec((2, HALF, H), lambda i: (0, i, 0)),
                 _row_spec((N, H))]
                + [_half_spec(s.shape) for s in gru_s]
                + [_half_spec(s.shape) for s in mlp_s]
                + [_full_spec((H, H)), _full_spec((H, H)), _full_spec((1, H))])
    out_specs = [_row_spec((N, H)), _row_spec((N, 2 * H)), _row_spec((N, 2 * H))]
    out_shape = [jax.ShapeDtypeStruct((N, H), jnp.float32),
                 jax.ShapeDtypeStruct((N, 2 * H), jnp.float32),
                 jax.ShapeDtypeStruct((N, 2 * H), jnp.float32)]
    return pl.pallas_call(
        _update_body, grid=(2,), in_specs=in_specs, out_specs=out_specs,
        out_shape=out_shape, interpret=interpret,
    )(agg2, h, *gru_s, *mlp_s, *att)


def _tc_final(agg2v, hv, gru_v, ga, act, val, interpret=False):
    out_shape = [jax.ShapeDtypeStruct((HALF, 4), jnp.float32),
                 jax.ShapeDtypeStruct((1, 1), jnp.float32)]
    return pl.pallas_call(
        _final_body, out_shape=out_shape, interpret=interpret,
    )(agg2v, hv, *gru_v, *ga, *act, *val)


def _gather_body(src_hbm, dst_hbm, a_hbm, mb_hbm, omb_hbm, oa_hbm,
                 sidx, didx, mb_v, a_v, sem1, sem2):
    cid = lax.axis_index("c")
    tid = lax.axis_index("s")
    wid = tid * NC + cid
    ept = src_hbm.shape[2]
    nblocks = ept // K

    def block_body(b, c):
        eb = b * K
        pltpu.sync_copy(src_hbm.at[wid, 0, pl.ds(eb, K)], sidx)
        pltpu.sync_copy(dst_hbm.at[wid, 0, pl.ds(eb, K)], didx)
        cp1 = pltpu.async_copy(mb_hbm.at[sidx], mb_v, sem1)
        cp2 = pltpu.async_copy(a_hbm.at[didx], a_v, sem2)
        cp1.wait()
        cp2.wait()
        rb = wid * ept + eb
        pltpu.sync_copy(mb_v, omb_hbm.at[pl.ds(rb, K)])
        pltpu.sync_copy(a_v, oa_hbm.at[pl.ds(rb, K)])
        return c
    lax.fori_loop(0, nblocks, block_body, 0)


def _sc_gather(src_p, dst_p, a_t, mb_t):
    ept = src_p.shape[2]
    epad = NW * ept
    mesh = plsc.VectorSubcoreMesh(core_axis_name="c", subcore_axis_name="s",
                                  num_cores=NC, num_subcores=NS)
    f = functools.partial(
        pl.kernel, _gather_body, mesh=mesh,
        compiler_params=pltpu.CompilerParams(needs_layout_passes=False),
        out_type=(jax.ShapeDtypeStruct((epad, 2 * H), jnp.float32),
                  jax.ShapeDtypeStruct((epad, 2 * H), jnp.float32)),
        scratch_types=[
            pltpu.VMEM((K,), jnp.int32),
            pltpu.VMEM((K,), jnp.int32),
            pltpu.VMEM((K, 2 * H), jnp.float32),
            pltpu.VMEM((K, 2 * H), jnp.float32),
            pltpu.SemaphoreType.DMA,
            pltpu.SemaphoreType.DMA,
        ],
    )()
    return f(src_p, dst_p, a_t, mb_t)


def _edge_tc_body(mb_r, a_r, s_r, d_r, w2_r, b2_r, con_r):
    pre = jax.nn.relu(a_r[:, H:] + mb_r[:, H:])
    logit = jnp.sum(pre * w2_r[...], axis=-1, keepdims=True) + b2_r[...]
    att = jax.nn.sigmoid(logit)
    valid = jnp.where((s_r[...] >= HALF) != (d_r[...] >= HALF), 1.0, 0.0)
    con_r[...] = mb_r[:, :H] * (att * valid)


def _tc_edge(mbr, ar, scol, dcol, w2row, b2s, be=4096):
    epad = mbr.shape[0]
    grid = (epad // be,)
    rs = lambda w: pl.BlockSpec((be, w), lambda i: (i, 0))
    return pl.pallas_call(
        _edge_tc_body, grid=grid,
        in_specs=[rs(2 * H), rs(2 * H), rs(1), rs(1),
                  _full_spec((1, H)), _full_spec((1, 1))],
        out_specs=rs(H),
        out_shape=jax.ShapeDtypeStruct((epad, H), jnp.float32),
    )(mbr, ar, scol, dcol, w2row, b2s)


def _scatter_body(dst_hbm, con_hbm, out_hbm, didx, con_v, zb_v, acc):
    cid = lax.axis_index("c")
    tid = lax.axis_index("s")
    wid = tid * NC + cid
    nblocks = dst_hbm.shape[1]
    ept = nblocks * K

    zrow = jnp.zeros((16,), jnp.float32)
    zrows = zb_v.shape[0]

    def zb_body(i, c):
        for j in range(H // 16):
            zb_v[i, pl.ds(j * 16, 16)] = zrow
        return c
    lax.fori_loop(0, zrows, zb_body, 0)
    base = tid * ROWS_PER_TILE
    for j in range(ROWS_PER_TILE // zrows):
        pltpu.sync_copy(zb_v, acc.at[pl.ds(base + j * zrows, zrows)])
    plsc.subcore_barrier()

    pltpu.sync_copy(dst_hbm.at[wid], didx)

    def block_body(b, c):
        pltpu.sync_copy(con_hbm.at[pl.ds(wid * ept + b * K, K)], con_v)
        pltpu.sync_copy(con_v, acc.at[didx.at[b]], add=True)
        return c
    lax.fori_loop(0, nblocks, block_body, 0)

    plsc.subcore_barrier()
    pltpu.sync_copy(acc.at[pl.ds(base, ROWS_PER_TILE)], out_hbm.at[cid, tid])


def _sc_scatter(dst_p, con):
    ept = dst_p.shape[2]
    nblocks = ept // K
    dst3 = dst_p.reshape(NW, nblocks, K)
    mesh = plsc.VectorSubcoreMesh(core_axis_name="c", subcore_axis_name="s",
                                  num_cores=NC, num_subcores=NS)
    f = functools.partial(
        pl.kernel, _scatter_body, mesh=mesh,
        compiler_params=pltpu.CompilerParams(needs_layout_passes=False),
        out_type=jax.ShapeDtypeStruct((2, NS, ROWS_PER_TILE, H), jnp.float32),
        scratch_types=[
            pltpu.VMEM((nblocks, K), jnp.int32),
            pltpu.VMEM((K, H), jnp.float32),
            pltpu.VMEM((25, H), jnp.float32),
            pltpu.VMEM_SHARED((N, H), jnp.float32),
        ],
    )()
    return f(dst3, con).reshape(2, N, H)


def _sc_edge(src_p, dst_p, a_t, mb_t, w2, b2bc):
    mbr, ar = _sc_gather(src_p, dst_p, a_t, mb_t)
    epad = mbr.shape[0]
    scol = src_p.reshape(epad, 1)
    dcol = dst_p.reshape(epad, 1)
    con = _tc_edge(mbr, ar, scol, dcol, w2.reshape(1, H), b2bc[:1].reshape(1, 1))
    agg = jnp.zeros((N, H), jnp.float32).at[dst_p.reshape(-1)].add(con)
    return jnp.stack([agg, jnp.zeros_like(agg)], axis=0)


def _stack(layers, *path):
    def get(lp):
        v = lp
        for p in path:
            v = v[p]
        return v
    return jnp.stack([get(lp) for lp in layers], axis=0)


def _row2(v):
    return v.reshape(1, -1)


def kernel(x, edge_index, node_type, params):
    del node_type
    e = edge_index.shape[1]
    ept = -(-e // (NW * K)) * K
    pad = NW * ept - e
    src_p = jnp.pad(edge_index[0], (0, pad)).reshape(NW, 1, ept)
    dst_p = jnp.pad(edge_index[1], (0, pad)).reshape(NW, 1, ept)

    lps = params['layers']
    emb = params['embed']
    emb_in = (_row2(emb['W'][:, 0]), _row2(emb['b']), _row2(emb['g']),
              _row2(emb['be']))

    def mlp_stack(l):
        pair = (lps[l]['c2v'], lps[l]['v2c'])
        out = []
        for k in ('W1', 'b1', 'g1', 'be1', 'W2', 'b2', 'g2', 'be2'):
            v = jnp.stack([p[k] for p in pair], axis=0)
            if v.ndim == 2:
                v = v[:, None, :]
            out.append(v)
        return out

    def att_in(l):
        a = lps[l]['att']
        w1 = a['W1']
        return (w1[:, :H], w1[:, H:], _row2(a['b1']))

    def gru_stack(l):
        pair = ((lps[l]['gru_check'], lps[l]['ln_check']),
                (lps[l]['gru_var'], lps[l]['ln_var']))
        out = []
        for k in ('Wi', 'bi', 'Wh', 'bh'):
            v = jnp.stack([p[0][k] for p in pair], axis=0)
            out.append(v[:, None, :] if v.ndim == 2 else v)
        out.append(jnp.stack([p[1]['g'] for p in pair], axis=0)[:, None, :])
        out.append(jnp.stack([p[1]['b'] for p in pair], axis=0)[:, None, :])
        return out

    h, a_t, mb_t = _tc_embed(x.reshape(N, 1), emb_in, mlp_stack(0), att_in(0))

    for l in range(len(lps)):
        att = lps[l]['att']
        w2 = att['W2'][0]
        b2bc = jnp.broadcast_to(att['b2'], (16,))
        agg2 = _sc_edge(src_p, dst_p, a_t, mb_t, w2, b2bc)
        if l + 1 < len(lps):
            h, a_t, mb_t = _tc_update(agg2, h, gru_stack(l),
                                      mlp_stack(l + 1), att_in(l + 1))
        else:
            gv = lps[l]['gru_var']
            lv = lps[l]['ln_var']
            gru_v = (gv['Wi'], _row2(gv['bi']), gv['Wh'], _row2(gv['bh']),
                     _row2(lv['g']), _row2(lv['b']))
            ga = params['glob_att']
            ga_in = (ga['W1'], _row2(ga['b1']), ga['W2'], ga['b2'].reshape(1, 1))
            ap = params['action']
            act_in = (ap['W1'], _row2(ap['b1']), _row2(ap['g']), _row2(ap['be']),
                      ap['W2'], _row2(ap['b2']))
            vp = params['value']
            val_in = (vp['W1'], _row2(vp['b1']), _row2(vp['g']), _row2(vp['be']),
                      vp['W2'], vp['b2'].reshape(1, 1))
            logits, value = _tc_final(agg2[:, HALF:], h[HALF:], gru_v,
                                      ga_in, act_in, val_in)
    return (logits, value)


# speedup vs baseline: 3.3090x; 1.2675x over previous
import functools

import jax
import jax.numpy as jnp
from jax import lax
from jax.experimental import pallas as pl
from jax.experimental.pallas import tpu as pltpu
from jax.experimental.pallas import tpu_sc as plsc

H = 64
HALF = 5000
N = 10000
NC = 2
NS = 16
NW = NC * NS
K = 128
GROUPS = K // 16
ROWS_PER_TILE = N // NS


def _lnk(x, g, b):
    m = jnp.mean(x, axis=-1, keepdims=True)
    v = jnp.mean((x - m) ** 2, axis=-1, keepdims=True)
    return (x - m) / jnp.sqrt(v + 1e-5) * g + b


def _dotT(x, w):
    return lax.dot_general(x, w, (((1,), (1,)), ((), ())),
                           preferred_element_type=jnp.float32)


def _gru_k(agg, h, wi, bi, wh, bh):
    gi = _dotT(agg, wi) + bi
    gh = _dotT(h, wh) + bh
    r = jax.nn.sigmoid(gi[:, :H] + gh[:, :H])
    z = jax.nn.sigmoid(gi[:, H:2 * H] + gh[:, H:2 * H])
    n = jnp.tanh(gi[:, 2 * H:] + r * gh[:, 2 * H:])
    return (1.0 - z) * n + z * h


def _mlp2_k(x, w1, b1, g1, be1, w2, b2, g2, be2):
    h1 = jax.nn.relu(_lnk(_dotT(x, w1) + b1, g1, be1))
    return _lnk(_dotT(h1, w2) + b2, g2, be2)


def _embed_body(x_r, wv_r, eb_r, eg_r, ebe_r,
                w1_r, b1_r, g1_r, be1_r, w2_r, b2_r, g2_r, be2_r,
                w1a_r, w1b_r, ab1_r,
                h_r, a_r, mb_r):
    pre = x_r[...] * wv_r[...] + eb_r[...]
    h0 = jax.nn.relu(_lnk(pre, eg_r[...], ebe_r[...]))
    m = _mlp2_k(h0, w1_r[0], b1_r[0], g1_r[0], be1_r[0],
                w2_r[0], b2_r[0], g2_r[0], be2_r[0])
    h_r[...] = h0
    a = _dotT(h0, w1a_r[...])
    a_r[:, :H] = a
    a_r[:, H:] = a
    mb_r[:, :H] = m
    mb_r[:, H:] = _dotT(m, w1b_r[...]) + ab1_r[...]


def _update_body(agg2_r, h_r,
                 wi_r, bi_r, wh_r, bh_r, lg_r, lb_r,
                 w1_r, b1_r, g1_r, be1_r, w2_r, b2_r, g2_r, be2_r,
                 w1a_r, w1b_r, ab1_r,
                 hn_r, a_r, mb_r):
    agg = agg2_r[0] + agg2_r[1]
    h = h_r[...]
    hn = _lnk(_gru_k(agg, h, wi_r[0], bi_r[0], wh_r[0], bh_r[0]) + h,
              lg_r[0], lb_r[0])
    m = _mlp2_k(hn, w1_r[0], b1_r[0], g1_r[0], be1_r[0],
                w2_r[0], b2_r[0], g2_r[0], be2_r[0])
    hn_r[...] = hn
    a = _dotT(hn, w1a_r[...])
    a_r[:, :H] = a
    a_r[:, H:] = a
    mb_r[:, :H] = m
    mb_r[:, H:] = _dotT(m, w1b_r[...]) + ab1_r[...]


def _final_body(agg2_r, h_r,
                wi_r, bi_r, wh_r, bh_r, lg_r, lb_r,
                gw1_r, gb1_r, gw2_r, gb2_r,
                aw1_r, ab1_r, ag_r, abe_r, aw2_r, ab2_r,
                vw1_r, vb1_r, vg_r, vbe_r, vw2_r, vb2_r,
                logits_r, value_r):
    agg = agg2_r[0] + agg2_r[1]
    h = h_r[...]
    vf = _lnk(_gru_k(agg, h, wi_r[...], bi_r[...], wh_r[...], bh_r[...]) + h,
              lg_r[...], lb_r[...])
    gw = jax.nn.sigmoid(
        jnp.sum(jax.nn.relu(_dotT(vf, gw1_r[...]) + gb1_r[...]) * gw2_r[...],
                axis=-1, keepdims=True) + gb2_r[...])
    glob = jnp.sum(vf * gw, axis=0, keepdims=True)
    ha = jax.nn.relu(_lnk(_dotT(vf, aw1_r[...]) + ab1_r[...],
                          ag_r[...], abe_r[...]))
    logits_r[...] = _dotT(ha, aw2_r[...]) + ab2_r[...]
    hv = jax.nn.relu(_lnk(_dotT(glob, vw1_r[...]) + vb1_r[...],
                          vg_r[...], vbe_r[...]))
    value_r[...] = (jnp.sum(hv * vw2_r[...], axis=-1, keepdims=True)
                    + vb2_r[...])


def _half_spec(shape):
    return pl.BlockSpec((1,) + shape[1:], lambda i: (i,) + (0,) * (len(shape) - 1))


def _full_spec(shape):
    return pl.BlockSpec(shape, lambda i: (0,) * len(shape))


def _row_spec(shape):
    return pl.BlockSpec((HALF,) + shape[1:], lambda i: (i,) + (0,) * (len(shape) - 1))


def _tc_embed(x2, emb, mlp_s, att, interpret=False):
    in_specs = ([_row_spec((N, 1))] + [_full_spec((1, H))] * 4
                + [_half_spec(s.shape) for s in mlp_s]
                + [_full_spec((H, H)), _full_spec((H, H)), _full_spec((1, H))])
    out_specs = [_row_spec((N, H)), _row_spec((N, 2 * H)), _row_spec((N, 2 * H))]
    out_shape = [jax.ShapeDtypeStruct((N, H), jnp.float32),
                 jax.ShapeDtypeStruct((N, 2 * H), jnp.float32),
                 jax.ShapeDtypeStruct((N, 2 * H), jnp.float32)]
    return pl.pallas_call(
        _embed_body, grid=(2,), in_specs=in_specs, out_specs=out_specs,
        out_shape=out_shape, interpret=interpret,
    )(x2, *emb, *mlp_s, *att)


def _tc_update(agg2, h, gru_s, mlp_s, att, interpret=False):
    in_specs = ([pl.BlockSpec((2, HALF, H), lambda i: (0, i, 0)),
                 _row_spec((N, H))]
                + [_half_spec(s.shape) for s in gru_s]
                + [_half_spec(s.shape) for s in mlp_s]
                + [_full_spec((H, H)), _full_spec((H, H)), _full_spec((1, H))])
    out_specs = [_row_spec((N, H)), _row_spec((N, 2 * H)), _row_spec((N, 2 * H))]
    out_shape = [jax.ShapeDtypeStruct((N, H), jnp.float32),
                 jax.ShapeDtypeStruct((N, 2 * H), jnp.float32),
                 jax.ShapeDtypeStruct((N, 2 * H), jnp.float32)]
    return pl.pallas_call(
        _update_body, grid=(2,), in_specs=in_specs, out_specs=out_specs,
        out_shape=out_shape, interpret=interpret,
    )(agg2, h, *gru_s, *mlp_s, *att)


def _tc_final(agg2v, hv, gru_v, ga, act, val, interpret=False):
    out_shape = [jax.ShapeDtypeStruct((HALF, 4), jnp.float32),
                 jax.ShapeDtypeStruct((1, 1), jnp.float32)]
    return pl.pallas_call(
        _final_body, out_shape=out_shape, interpret=interpret,
    )(agg2v, hv, *gru_v, *ga, *act, *val)


def _gather_body(src_hbm, dst_hbm, a_hbm, mb_hbm, omb_hbm, oa_hbm,
                 sidx, didx, mb_v, a_v, sem1, sem2):
    cid = lax.axis_index("c")
    tid = lax.axis_index("s")
    wid = tid * NC + cid
    ept = src_hbm.shape[2]
    nblocks = ept // K

    def block_body(b, c):
        eb = b * K
        pltpu.sync_copy(src_hbm.at[wid, 0, pl.ds(eb, K)], sidx)
        pltpu.sync_copy(dst_hbm.at[wid, 0, pl.ds(eb, K)], didx)
        cp1 = pltpu.async_copy(mb_hbm.at[sidx], mb_v, sem1)
        cp2 = pltpu.async_copy(a_hbm.at[didx], a_v, sem2)
        cp1.wait()
        cp2.wait()
        rb = wid * ept + eb
        pltpu.sync_copy(mb_v, omb_hbm.at[pl.ds(rb, K)])
        pltpu.sync_copy(a_v, oa_hbm.at[pl.ds(rb, K)])
        return c
    lax.fori_loop(0, nblocks, block_body, 0)


def _sc_gather(src_p, dst_p, a_t, mb_t):
    ept = src_p.shape[2]
    epad = NW * ept
    mesh = plsc.VectorSubcoreMesh(core_axis_name="c", subcore_axis_name="s",
                                  num_cores=NC, num_subcores=NS)
    f = functools.partial(
        pl.kernel, _gather_body, mesh=mesh,
        compiler_params=pltpu.CompilerParams(needs_layout_passes=False),
        out_type=(jax.ShapeDtypeStruct((epad, 2 * H), jnp.float32),
                  jax.ShapeDtypeStruct((epad, 2 * H), jnp.float32)),
        scratch_types=[
            pltpu.VMEM((K,), jnp.int32),
            pltpu.VMEM((K,), jnp.int32),
            pltpu.VMEM((K, 2 * H), jnp.float32),
            pltpu.VMEM((K, 2 * H), jnp.float32),
            pltpu.SemaphoreType.DMA,
            pltpu.SemaphoreType.DMA,
        ],
    )()
    return f(src_p, dst_p, a_t, mb_t)


def _edge_tc_body(mb_r, a_r, s_r, d_r, w2_r, b2_r, con_r):
    pre = jax.nn.relu(a_r[:, H:] + mb_r[:, H:])
    logit = jnp.sum(pre * w2_r[...], axis=-1, keepdims=True) + b2_r[...]
    att = jax.nn.sigmoid(logit)
    valid = jnp.where((s_r[...] >= HALF) != (d_r[...] >= HALF), 1.0, 0.0)
    con_r[...] = mb_r[...] * (att * valid)


def _tc_edge(mbr, ar, scol, dcol, w2row, b2s, be=4096):
    epad = mbr.shape[0]
    grid = (epad // be,)
    rs = lambda w: pl.BlockSpec((be, w), lambda i: (i, 0))
    return pl.pallas_call(
        _edge_tc_body, grid=grid,
        in_specs=[rs(2 * H), rs(2 * H), rs(1), rs(1),
                  _full_spec((1, H)), _full_spec((1, 1))],
        out_specs=rs(2 * H),
        out_shape=jax.ShapeDtypeStruct((epad, 2 * H), jnp.float32),
    )(mbr, ar, scol, dcol, w2row, b2s)


def _scatter_body(dst_hbm, con_hbm, out_hbm, didx, con_v, zb_v, acc):
    cid = lax.axis_index("c")
    tid = lax.axis_index("s")
    wid = tid * NC + cid
    nblocks = dst_hbm.shape[1]
    ept = nblocks * K

    zrow = jnp.zeros((16,), jnp.float32)
    zrows = zb_v.shape[0]

    def zb_body(i, c):
        for j in range(2 * H // 16):
            zb_v[i, pl.ds(j * 16, 16)] = zrow
        return c
    lax.fori_loop(0, zrows, zb_body, 0)
    base = tid * ROWS_PER_TILE
    for j in range(ROWS_PER_TILE // zrows):
        pltpu.sync_copy(zb_v, acc.at[pl.ds(base + j * zrows, zrows)])
    plsc.subcore_barrier()

    pltpu.sync_copy(dst_hbm.at[wid], didx)

    def block_body(b, c):
        pltpu.sync_copy(con_hbm.at[pl.ds(wid * ept + b * K, K)], con_v)
        pltpu.sync_copy(con_v, acc.at[didx.at[b]], add=True)
        return c
    lax.fori_loop(0, nblocks, block_body, 0)

    plsc.subcore_barrier()
    pltpu.sync_copy(acc.at[pl.ds(base, ROWS_PER_TILE)], out_hbm.at[cid, tid])


def _sc_scatter(dst_p, con):
    ept = dst_p.shape[2]
    nblocks = ept // K
    dst3 = dst_p.reshape(NW, nblocks, K)
    mesh = plsc.VectorSubcoreMesh(core_axis_name="c", subcore_axis_name="s",
                                  num_cores=NC, num_subcores=NS)
    f = functools.partial(
        pl.kernel, _scatter_body, mesh=mesh,
        compiler_params=pltpu.CompilerParams(needs_layout_passes=False),
        out_type=jax.ShapeDtypeStruct((2, NS, ROWS_PER_TILE, 2 * H), jnp.float32),
        scratch_types=[
            pltpu.VMEM((nblocks, K), jnp.int32),
            pltpu.VMEM((K, 2 * H), jnp.float32),
            pltpu.VMEM((25, 2 * H), jnp.float32),
            pltpu.VMEM_SHARED((N, 2 * H), jnp.float32),
        ],
    )()
    return f(dst3, con).reshape(2, N, 2 * H)[:, :, :H]


def _sc_edge(src_p, dst_p, a_t, mb_t, w2, b2bc):
    mbr, ar = _sc_gather(src_p, dst_p, a_t, mb_t)
    epad = mbr.shape[0]
    scol = src_p.reshape(epad, 1)
    dcol = dst_p.reshape(epad, 1)
    con = _tc_edge(mbr, ar, scol, dcol, w2.reshape(1, H), b2bc[:1].reshape(1, 1))
    return _sc_scatter(dst_p, con)


def _stack(layers, *path):
    def get(lp):
        v = lp
        for p in path:
            v = v[p]
        return v
    return jnp.stack([get(lp) for lp in layers], axis=0)


def _row2(v):
    return v.reshape(1, -1)


def kernel(x, edge_index, node_type, params):
    del node_type
    e = edge_index.shape[1]
    ept = -(-e // (NW * K)) * K
    pad = NW * ept - e
    src_p = jnp.pad(edge_index[0], (0, pad)).reshape(NW, 1, ept)
    dst_p = jnp.pad(edge_index[1], (0, pad)).reshape(NW, 1, ept)

    lps = params['layers']
    emb = params['embed']
    emb_in = (_row2(emb['W'][:, 0]), _row2(emb['b']), _row2(emb['g']),
              _row2(emb['be']))

    def mlp_stack(l):
        pair = (lps[l]['c2v'], lps[l]['v2c'])
        out = []
        for k in ('W1', 'b1', 'g1', 'be1', 'W2', 'b2', 'g2', 'be2'):
            v = jnp.stack([p[k] for p in pair], axis=0)
            if v.ndim == 2:
                v = v[:, None, :]
            out.append(v)
        return out

    def att_in(l):
        a = lps[l]['att']
        w1 = a['W1']
        return (w1[:, :H], w1[:, H:], _row2(a['b1']))

    def gru_stack(l):
        pair = ((lps[l]['gru_check'], lps[l]['ln_check']),
                (lps[l]['gru_var'], lps[l]['ln_var']))
        out = []
        for k in ('Wi', 'bi', 'Wh', 'bh'):
            v = jnp.stack([p[0][k] for p in pair], axis=0)
            out.append(v[:, None, :] if v.ndim == 2 else v)
        out.append(jnp.stack([p[1]['g'] for p in pair], axis=0)[:, None, :])
        out.append(jnp.stack([p[1]['b'] for p in pair], axis=0)[:, None, :])
        return out

    h, a_t, mb_t = _tc_embed(x.reshape(N, 1), emb_in, mlp_stack(0), att_in(0))

    for l in range(len(lps)):
        att = lps[l]['att']
        w2 = att['W2'][0]
        b2bc = jnp.broadcast_to(att['b2'], (16,))
        agg2 = _sc_edge(src_p, dst_p, a_t, mb_t, w2, b2bc)
        if l + 1 < len(lps):
            h, a_t, mb_t = _tc_update(agg2, h, gru_stack(l),
                                      mlp_stack(l + 1), att_in(l + 1))
        else:
            gv = lps[l]['gru_var']
            lv = lps[l]['ln_var']
            gru_v = (gv['Wi'], _row2(gv['bi']), gv['Wh'], _row2(gv['bh']),
                     _row2(lv['g']), _row2(lv['b']))
            ga = params['glob_att']
            ga_in = (ga['W1'], _row2(ga['b1']), ga['W2'], ga['b2'].reshape(1, 1))
            ap = params['action']
            act_in = (ap['W1'], _row2(ap['b1']), _row2(ap['g']), _row2(ap['be']),
                      ap['W2'], _row2(ap['b2']))
            vp = params['value']
            val_in = (vp['W1'], _row2(vp['b1']), _row2(vp['g']), _row2(vp['be']),
                      vp['W2'], vp['b2'].reshape(1, 1))
            logits, value = _tc_final(agg2[:, HALF:], h[HALF:], gru_v,
                                      ga_in, act_in, val_in)
    return (logits, value)
